# Initial kernel scaffold; baseline (speedup 1.0000x reference)
import jax
import jax.numpy as jnp
from jax.experimental import pallas as pl


def kernel(input, input_length, params):
    raise NotImplementedError("write your pallas kernel here")


# speedup vs baseline: 1.2482x; 1.2482x over previous
import jax
import jax.numpy as jnp
from jax.experimental import pallas as pl
from jax.experimental.pallas import tpu as pltpu


def _mm(a, b):
    return jnp.dot(a, b, preferred_element_type=jnp.float32)


def _mm_t(a, b):
    return jax.lax.dot_general(a, b, (((1,), (1,)), ((), ())),
                               preferred_element_type=jnp.float32)


def _enc_block(feat, W1, b1, W2, b2, mask):
    h = jnp.maximum(_mm(feat, W1) + b1, 0.0)
    h = _mm(h, W2) + b2
    s = feat + h
    m = jnp.mean(s, axis=-1, keepdims=True)
    d = s - m
    v = jnp.mean(d * d, axis=-1, keepdims=True)
    return (d / jnp.sqrt(v + 1e-5)) * mask


def _vq(qin, cb_ref):
    qs, idxs = [], []
    dsum = jnp.float32(0.0)
    for h in range(4):
        z = qin[:, 64 * h:64 * (h + 1)]
        c = cb_ref[h]
        cn = jnp.sum(c * c, axis=-1)[None, :]
        zn = jnp.sum(z * z, axis=-1, keepdims=True)
        d2 = zn - 2.0 * _mm_t(z, c) + cn
        mn = jnp.min(d2, axis=-1, keepdims=True)
        kio = jax.lax.broadcasted_iota(jnp.int32, d2.shape, 1)
        idx = jnp.min(jnp.where(d2 <= mn, kio, jnp.int32(1 << 30)),
                      axis=-1, keepdims=True)
        oh = (kio == idx).astype(jnp.float32)
        q = _mm(oh, c)
        e = q - z
        dsum = dsum + jnp.sum(e * e)
        qs.append(q)
        idxs.append(idx)
    return jnp.concatenate(qs, -1), jnp.concatenate(idxs, -1), dsum


def _body(len_ref, x_ref,
          e0W1, e0b1, e0W2, e0b2, e1W1, e1b1, e1W2, e1b2,
          p0Wa, p0ba, p0Wb, p0bb, cb0,
          q0W1, q0b1, q0W2, q0b2,
          prW1, prb1, prW2, prb2,
          p1WaE, p1WaR, p1ba, p1Wb, p1bb, cb1,
          q1W1R, q1W1Q, q1b1, q1W2, q1b2,
          res_ref, idx0_ref, idx1_ref, stats_ref):
    b = pl.program_id(0)
    L = len_ref[b]
    x = x_ref[0]

    io_t = jax.lax.broadcasted_iota(jnp.int32, (1024, 1), 0)
    mask0 = (io_t < L).astype(jnp.float32)
    L1 = (L + 3) // 4
    io_s = jax.lax.broadcasted_iota(jnp.int32, (256, 1), 0)
    mask1 = (io_s < L1).astype(jnp.float32)

    feat0 = _enc_block(x, e0W1[...], e0b1[...], e0W2[...], e0b2[...], mask0)
    pooled = jnp.mean(feat0.reshape(256, 4, 256), axis=1)
    feat1 = _enc_block(pooled, e1W1[...], e1b1[...], e1W2[...], e1b2[...],
                       mask1)

    t = jnp.tanh(_mm(feat1, p0Wa[...]) + p0ba[...])
    qin0 = _mm(t, p0Wb[...]) + p0bb[...]
    q0, idx0, d0sum = _vq(qin0, cb0)
    quant0 = q0 * mask1
    post = jnp.tanh(_mm(quant0, q0W1[...]) + q0b1[...])
    post = _mm(post, q0W2[...]) + q0b2[...]
    residual = jnp.broadcast_to(post[:, None, :],
                                (256, 4, 256)).reshape(1024, 256)

    h = jnp.tanh(_mm(residual, prW1[...]) + prb1[...])
    pred_hidden = (residual + h) * mask0
    pred_quant = (_mm(pred_hidden, prW2[...]) + prb2[...]) * mask0
    residual = residual + pred_hidden
    t = jnp.tanh(_mm(feat0, p1WaE[...]) + _mm(residual, p1WaR[...])
                 + p1ba[...])
    qin1 = _mm(t, p1Wb[...]) + p1bb[...]
    q1, idx1, d1sum = _vq(qin1, cb1)
    quant1 = q1 * mask0
    pe = pred_quant - quant1
    psum = jnp.sum(pe * pe)
    post = jnp.tanh(_mm(residual, q1W1R[...]) + _mm(quant1, q1W1Q[...])
                    + q1b1[...])
    post = _mm(post, q1W2[...]) + q1b2[...]
    residual = residual + post

    res_ref[0] = residual
    idx0_ref[0] = idx0
    idx1_ref[0] = idx1
    lane = jax.lax.broadcasted_iota(jnp.int32, (1, 128), 1)
    stats_ref[0] = jnp.where(
        lane == 0, d0sum,
        jnp.where(lane == 1, d1sum, jnp.where(lane == 2, psum, 0.0)))


@jax.jit
def kernel(input, input_length, params):
    p = params
    r1 = lambda a: a.reshape(1, -1)
    plist = [
        p['enc0_W1'], r1(p['enc0_b1']), p['enc0_W2'], r1(p['enc0_b2']),
        p['enc1_W1'], r1(p['enc1_b1']), p['enc1_W2'], r1(p['enc1_b2']),
        p['pre0_Wa'], r1(p['pre0_ba']), p['pre0_Wb'], r1(p['pre0_bb']),
        p['cb0'],
        p['post0_W1'], r1(p['post0_b1']), p['post0_W2'], r1(p['post0_b2']),
        p['prd1_W1'], r1(p['prd1_b1']), p['prd1_W2'], r1(p['prd1_b2']),
        p['pre1_Wa'][:256], p['pre1_Wa'][256:], r1(p['pre1_ba']),
        p['pre1_Wb'], r1(p['pre1_bb']),
        p['cb1'],
        p['post1_W1'][:256], p['post1_W1'][256:], r1(p['post1_b1']),
        p['post1_W2'], r1(p['post1_b2']),
    ]

    def _full(a):
        nd = a.ndim
        return pl.BlockSpec(a.shape, lambda b, L, _n=nd: (0,) * _n)

    grid_spec = pltpu.PrefetchScalarGridSpec(
        num_scalar_prefetch=1,
        grid=(8,),
        in_specs=[pl.BlockSpec((1, 1024, 256), lambda b, L: (b, 0, 0))]
                 + [_full(a) for a in plist],
        out_specs=[
            pl.BlockSpec((1, 1024, 256), lambda b, L: (b, 0, 0)),
            pl.BlockSpec((1, 256, 4), lambda b, L: (b, 0, 0)),
            pl.BlockSpec((1, 1024, 4), lambda b, L: (b, 0, 0)),
            pl.BlockSpec((1, 1, 128), lambda b, L: (b, 0, 0)),
        ],
    )
    res, idx0, idx1, stats = pl.pallas_call(
        _body,
        grid_spec=grid_spec,
        out_shape=[
            jax.ShapeDtypeStruct((8, 1024, 256), jnp.float32),
            jax.ShapeDtypeStruct((8, 256, 4), jnp.int32),
            jax.ShapeDtypeStruct((8, 1024, 4), jnp.int32),
            jax.ShapeDtypeStruct((8, 1, 128), jnp.float32),
        ],
        compiler_params=pltpu.CompilerParams(
            dimension_semantics=("arbitrary",)),
    )(input_length, input, *plist)

    d0 = jnp.sum(stats[:, 0, 0]) / (8 * 256 * 256)
    d1 = jnp.sum(stats[:, 0, 1]) / (8 * 1024 * 256)
    pp = jnp.sum(stats[:, 0, 2]) / (8 * 1024 * 256)
    total_diff = d0 + d1 + pp
    return res, total_diff, (idx0, idx1)
